# baseline (speedup 1.0000x reference)
import jax
import jax.numpy as jnp
from jax.experimental import pallas as pl
from jax.experimental.pallas import tpu as pltpu

D_IN = 768
D_SAE = 16384
K = 100
N_TOK = 8192

ROW_BLK = 256
FEAT_BLK = 2048


def _enc_block(x_ref, w_ref, b_ref, o_ref):
    acc = jnp.dot(x_ref[...], w_ref[...], preferred_element_type=jnp.float32)
    o_ref[...] = jnp.maximum(acc + b_ref[...][None, :], 0.0)


def _dec_block(e_ref, w_ref, b_ref, o_ref):
    k = pl.program_id(1)

    @pl.when(k == 0)
    def _():
        o_ref[...] = jnp.broadcast_to(b_ref[...][None, :], o_ref.shape)

    o_ref[...] += jnp.dot(e_ref[...], w_ref[...], preferred_element_type=jnp.float32)


def _encode(x, W_enc, b_enc):
    n, d_in = x.shape
    d_sae = W_enc.shape[1]
    grid = (n // ROW_BLK, d_sae // FEAT_BLK)
    return pl.pallas_call(
        _enc_block,
        grid=grid,
        in_specs=[
            pl.BlockSpec((ROW_BLK, d_in), lambda i, j: (i, 0)),
            pl.BlockSpec((d_in, FEAT_BLK), lambda i, j: (0, j)),
            pl.BlockSpec((FEAT_BLK,), lambda i, j: (j,)),
        ],
        out_specs=pl.BlockSpec((ROW_BLK, FEAT_BLK), lambda i, j: (i, j)),
        out_shape=jax.ShapeDtypeStruct((n, d_sae), jnp.float32),
        compiler_params=pltpu.CompilerParams(
            dimension_semantics=("parallel", "parallel"),
        ),
    )(x, W_enc, b_enc)


def _decode(encoded, W_dec, b_dec):
    n, d_sae = encoded.shape
    d_in = W_dec.shape[1]
    grid = (n // ROW_BLK, d_sae // FEAT_BLK)
    return pl.pallas_call(
        _dec_block,
        grid=grid,
        in_specs=[
            pl.BlockSpec((ROW_BLK, FEAT_BLK), lambda i, k: (i, k)),
            pl.BlockSpec((FEAT_BLK, d_in), lambda i, k: (k, 0)),
            pl.BlockSpec((d_in,), lambda i, k: (0,)),
        ],
        out_specs=pl.BlockSpec((ROW_BLK, d_in), lambda i, k: (i, 0)),
        out_shape=jax.ShapeDtypeStruct((n, d_in), jnp.float32),
        compiler_params=pltpu.CompilerParams(
            dimension_semantics=("parallel", "arbitrary"),
        ),
    )(encoded, W_dec, b_dec)


def kernel(x, W_enc, b_enc, W_dec, b_dec):
    acts = _encode(x - b_dec[None, :], W_enc, b_enc)
    top_vals, top_idx = jax.lax.top_k(acts, K)
    rows = jnp.arange(x.shape[0])[:, None]
    encoded = jnp.zeros_like(acts).at[rows, top_idx].set(top_vals)
    return _decode(encoded, W_dec, b_dec)


# speedup vs baseline: 29.9676x; 29.9676x over previous
import jax
import jax.numpy as jnp
from jax import lax
from jax.experimental import pallas as pl
from jax.experimental.pallas import tpu as pltpu

D_IN = 768
D_SAE = 16384
K = 100
N_TOK = 8192

ROW_BLK = 512
FEAT_BLK = 1024
N_CHUNK = D_SAE // FEAT_BLK
BISECT_ITERS = 22


def _fused_block(x_ref, we_ref, be_ref, wd_ref, bd_ref, o_ref, acts_s, thr_s):
    j = pl.program_id(1)

    @pl.when(j < N_CHUNK)
    def _encode():
        acc = jnp.dot(x_ref[...], we_ref[...], preferred_element_type=jnp.float32)
        acts_s[:, pl.ds(j * FEAT_BLK, FEAT_BLK)] = jnp.maximum(
            acc + be_ref[...][None, :], 0.0
        )

    @pl.when(j == N_CHUNK)
    def _bisect():
        a = acts_s[...]
        hi0 = jnp.max(a, axis=1, keepdims=True)
        lo0 = jnp.zeros_like(hi0)

        def body(_, carry):
            lo, hi = carry
            mid = 0.5 * (lo + hi)
            cnt = jnp.sum(
                jnp.where(acts_s[...] > mid, 1.0, 0.0), axis=1, keepdims=True
            )
            take = cnt >= K
            return jnp.where(take, mid, lo), jnp.where(take, hi, mid)

        lo, _ = lax.fori_loop(0, BISECT_ITERS, body, (lo0, hi0))
        thr_s[...] = lo

    @pl.when(j > N_CHUNK)
    def _decode():
        c = j - (N_CHUNK + 1)

        @pl.when(c == 0)
        def _():
            o_ref[...] = jnp.broadcast_to(bd_ref[...][None, :], o_ref.shape)

        e = acts_s[:, pl.ds(c * FEAT_BLK, FEAT_BLK)]
        e = jnp.where(e > thr_s[...], e, 0.0)
        o_ref[...] += jnp.dot(e, wd_ref[...], preferred_element_type=jnp.float32)


def kernel(x, W_enc, b_enc, W_dec, b_dec):
    n, d_in = x.shape
    d_sae = W_enc.shape[1]
    xc = x - b_dec[None, :]
    grid = (n // ROW_BLK, 2 * N_CHUNK + 1)

    def enc_chunk(i, j):
        return (0, jnp.minimum(j, N_CHUNK - 1))

    def dec_chunk(i, j):
        return (jnp.clip(j - (N_CHUNK + 1), 0, N_CHUNK - 1), 0)

    return pl.pallas_call(
        _fused_block,
        grid=grid,
        in_specs=[
            pl.BlockSpec((ROW_BLK, d_in), lambda i, j: (i, 0)),
            pl.BlockSpec((d_in, FEAT_BLK), enc_chunk),
            pl.BlockSpec((FEAT_BLK,), lambda i, j: (jnp.minimum(j, N_CHUNK - 1),)),
            pl.BlockSpec((FEAT_BLK, d_in), dec_chunk),
            pl.BlockSpec((d_in,), lambda i, j: (0,)),
        ],
        out_specs=pl.BlockSpec((ROW_BLK, d_in), lambda i, j: (i, 0)),
        out_shape=jax.ShapeDtypeStruct((n, d_in), jnp.float32),
        scratch_shapes=[
            pltpu.VMEM((ROW_BLK, d_sae), jnp.float32),
            pltpu.VMEM((ROW_BLK, 1), jnp.float32),
        ],
        compiler_params=pltpu.CompilerParams(
            dimension_semantics=("parallel", "arbitrary"),
        ),
    )(xc, W_enc, b_enc, W_dec, b_dec)


# speedup vs baseline: 30.5757x; 1.0203x over previous
import jax
import jax.numpy as jnp
from jax import lax
from jax.experimental import pallas as pl
from jax.experimental.pallas import tpu as pltpu

D_IN = 768
D_SAE = 16384
K = 100
N_TOK = 8192

ROW_BLK = 512
FEAT_BLK = 1024
N_CHUNK = D_SAE // FEAT_BLK
BISECT_ITERS = 22


def _fused_block(x_ref, we_ref, be_ref, wd_ref, bd_ref, o_ref, acts_s, thr_s):
    j = pl.program_id(1)

    @pl.when(j < N_CHUNK)
    def _encode():
        acc = jnp.dot(x_ref[...], we_ref[...], preferred_element_type=jnp.float32)
        acts_s[:, pl.ds(j * FEAT_BLK, FEAT_BLK)] = jnp.maximum(
            acc + be_ref[...][None, :], 0.0
        )

    @pl.when(j == N_CHUNK)
    def _bisect():
        a = acts_s[...]
        hi0 = jnp.max(a, axis=1, keepdims=True)
        lo0 = jnp.zeros_like(hi0)

        def body(_, carry):
            lo, hi = carry
            mid = 0.5 * (lo + hi)
            cnt = jnp.sum(
                jnp.where(acts_s[...] > mid, 1.0, 0.0), axis=1, keepdims=True
            )
            take = cnt >= K
            return jnp.where(take, mid, lo), jnp.where(take, hi, mid)

        lo, _ = lax.fori_loop(0, BISECT_ITERS, body, (lo0, hi0))
        thr_s[...] = lo

    @pl.when(j > N_CHUNK)
    def _decode():
        c = j - (N_CHUNK + 1)

        @pl.when(c == 0)
        def _():
            o_ref[...] = jnp.broadcast_to(bd_ref[...][None, :], o_ref.shape)

        e = acts_s[:, pl.ds(c * FEAT_BLK, FEAT_BLK)]
        e = jnp.where(e > thr_s[...], e, 0.0).astype(jnp.bfloat16)
        o_ref[...] += jnp.dot(e, wd_ref[...], preferred_element_type=jnp.float32)


def kernel(x, W_enc, b_enc, W_dec, b_dec):
    n, d_in = x.shape
    d_sae = W_enc.shape[1]
    xc = x - b_dec[None, :]
    wd_bf = W_dec.astype(jnp.bfloat16)
    grid = (n // ROW_BLK, 2 * N_CHUNK + 1)

    def enc_chunk(i, j):
        return (0, jnp.minimum(j, N_CHUNK - 1))

    def dec_chunk(i, j):
        return (jnp.clip(j - (N_CHUNK + 1), 0, N_CHUNK - 1), 0)

    return pl.pallas_call(
        _fused_block,
        grid=grid,
        in_specs=[
            pl.BlockSpec((ROW_BLK, d_in), lambda i, j: (i, 0)),
            pl.BlockSpec((d_in, FEAT_BLK), enc_chunk),
            pl.BlockSpec((FEAT_BLK,), lambda i, j: (jnp.minimum(j, N_CHUNK - 1),)),
            pl.BlockSpec((FEAT_BLK, d_in), dec_chunk),
            pl.BlockSpec((d_in,), lambda i, j: (0,)),
        ],
        out_specs=pl.BlockSpec((ROW_BLK, d_in), lambda i, j: (i, 0)),
        out_shape=jax.ShapeDtypeStruct((n, d_in), jnp.float32),
        scratch_shapes=[
            pltpu.VMEM((ROW_BLK, d_sae), jnp.float32),
            pltpu.VMEM((ROW_BLK, 1), jnp.float32),
        ],
        compiler_params=pltpu.CompilerParams(
            dimension_semantics=("parallel", "arbitrary"),
        ),
    )(xc, W_enc, b_enc, wd_bf, b_dec)
